# baseline (speedup 1.0000x reference)
import functools

import jax
import jax.numpy as jnp
from jax import lax
from jax.experimental import pallas as pl
from jax.experimental.pallas import tpu as pltpu
from jax.experimental.pallas import tpu_sc as plsc

N = 2048
D = 1024
E = 8
K = 2
BM = 256
NT = (N * K) // BM + E
P = NT * BM
NEG = -1e30


RB = 256
NB = N // RB


def _routing_body(x_ref, wcat_ref, bcat_ref, noise_ref,
                  gating_ref, dst0_ref, dst1_ref, w0r_ref, w1r_ref,
                  eot_ref, tval_ref,
                  rank_s, i0_s, i1_s, g0_s, g1_s, run_s, roff_s):
    s = pl.program_id(0)

    @pl.when(s == 0)
    def _init():
        run_s[...] = jnp.zeros((1, E), jnp.float32)

    @pl.when(s < NB)
    def _phase1():
        x = x_ref[...]
        z = jnp.dot(x, wcat_ref[...], preferred_element_type=jnp.float32)
        b = bcat_ref[...]
        logits = z[:, :E] + b[:, :E]
        nlog = z[:, E:] + b[:, E:]
        sp = jnp.maximum(nlog, 0.0) + jnp.log1p(jnp.exp(-jnp.abs(nlog)))
        noisy = logits + noise_ref[...] * sp

        iota8 = lax.broadcasted_iota(jnp.int32, (RB, E), 1)
        v0 = jnp.max(noisy, axis=1, keepdims=True)
        c0 = jnp.where(noisy >= v0, iota8, E)
        i0 = jnp.min(c0, axis=1, keepdims=True)
        oh0 = iota8 == i0
        masked = jnp.where(oh0, NEG, noisy)
        v1 = jnp.max(masked, axis=1, keepdims=True)
        c1 = jnp.where(masked >= v1, iota8, E)
        i1 = jnp.min(c1, axis=1, keepdims=True)
        oh1 = iota8 == i1

        e1 = jnp.exp(v1 - v0)
        denom = 1.0 + e1
        g0 = 1.0 / denom
        g1 = e1 / denom
        gating_ref[...] = jnp.where(oh0, g0, 0.0) + jnp.where(oh1, g1, 0.0)

        maskf = (oh0 | oh1).astype(jnp.float32)
        tri = (lax.broadcasted_iota(jnp.int32, (RB, RB), 1)
               < lax.broadcasted_iota(jnp.int32, (RB, RB), 0)).astype(jnp.float32)
        running = run_s[...]
        rs = pl.ds(s * RB, RB)
        rank_s[rs, :] = jnp.dot(tri, maskf,
                                preferred_element_type=jnp.float32) + running
        run_s[...] = running + jnp.sum(maskf, axis=0, keepdims=True)
        i0_s[rs, :] = i0
        i1_s[rs, :] = i1
        g0_s[rs, :] = g0
        g1_s[rs, :] = g1

    @pl.when(s == NB)
    def _totals():
        counts = run_s[...].astype(jnp.int32)
        tile_cnt = (counts + (BM - 1)) // BM
        lane = lax.broadcasted_iota(jnp.int32, (1, E), 1)
        tile_off = jnp.zeros((1, E), jnp.int32)
        for e in range(1, E):
            prev = jnp.sum(jnp.where(lane == e - 1, tile_cnt, 0),
                           axis=1, keepdims=True)
            tile_off = tile_off + jnp.where(lane >= e, prev, 0)
        roff_s[...] = (tile_off * BM).astype(jnp.float32)
        ends = tile_off + tile_cnt
        tt = lax.broadcasted_iota(jnp.int32, (NT, E), 0)
        eotv = jnp.sum((tt >= jnp.broadcast_to(ends, (NT, E))).astype(jnp.int32),
                       axis=1, keepdims=True)
        eot_ref[...] = jnp.minimum(eotv, E - 1)
        tval_ref[...] = (eotv < E).astype(jnp.int32)

    @pl.when(s >= NB)
    def _phase2():
        rs = pl.ds((s - NB) * RB, RB)
        iota8 = lax.broadcasted_iota(jnp.int32, (RB, E), 1)
        oh0 = iota8 == i0_s[rs, :]
        oh1 = iota8 == i1_s[rs, :]
        dstpos = jnp.broadcast_to(roff_s[...], (RB, E)) + rank_s[rs, :]
        dst0_ref[...] = jnp.sum(jnp.where(oh0, dstpos, 0.0), axis=1,
                                keepdims=True).astype(jnp.int32)
        dst1_ref[...] = jnp.sum(jnp.where(oh1, dstpos, 0.0), axis=1,
                                keepdims=True).astype(jnp.int32)
        w0r_ref[...] = jnp.broadcast_to(g0_s[rs, :], (RB, 128))
        w1r_ref[...] = jnp.broadcast_to(g1_s[rs, :], (RB, 128))


def _p1_idx(s):
    return jnp.where(s < NB, s, NB - 1)


def _p2_idx(s):
    return jnp.where(s < NB, 0, s - NB)


def _routing(x, wcat, bcat, noise):
    return pl.pallas_call(
        _routing_body,
        grid=(2 * NB,),
        in_specs=[
            pl.BlockSpec((RB, D), lambda s: (_p1_idx(s), 0)),
            pl.BlockSpec((D, 2 * E), lambda s: (0, 0)),
            pl.BlockSpec((1, 2 * E), lambda s: (0, 0)),
            pl.BlockSpec((RB, E), lambda s: (_p1_idx(s), 0)),
        ],
        out_specs=(
            pl.BlockSpec((RB, E), lambda s: (_p1_idx(s), 0)),
            pl.BlockSpec((RB, 1), lambda s: (_p2_idx(s), 0)),
            pl.BlockSpec((RB, 1), lambda s: (_p2_idx(s), 0)),
            pl.BlockSpec((RB, 128), lambda s: (_p2_idx(s), 0)),
            pl.BlockSpec((RB, 128), lambda s: (_p2_idx(s), 0)),
            pl.BlockSpec((NT, 1), lambda s: (0, 0)),
            pl.BlockSpec((NT, 1), lambda s: (0, 0)),
        ),
        out_shape=(
            jax.ShapeDtypeStruct((N, E), jnp.float32),
            jax.ShapeDtypeStruct((N, 1), jnp.int32),
            jax.ShapeDtypeStruct((N, 1), jnp.int32),
            jax.ShapeDtypeStruct((N, 128), jnp.float32),
            jax.ShapeDtypeStruct((N, 128), jnp.float32),
            jax.ShapeDtypeStruct((NT, 1), jnp.int32),
            jax.ShapeDtypeStruct((NT, 1), jnp.int32),
        ),
        scratch_shapes=[
            pltpu.VMEM((N, E), jnp.float32),
            pltpu.VMEM((N, 1), jnp.int32),
            pltpu.VMEM((N, 1), jnp.int32),
            pltpu.VMEM((N, 1), jnp.float32),
            pltpu.VMEM((N, 1), jnp.float32),
            pltpu.VMEM((1, E), jnp.float32),
            pltpu.VMEM((1, E), jnp.float32),
        ],
        compiler_params=pltpu.CompilerParams(
            dimension_semantics=("arbitrary",)),
    )(x, wcat, bcat, noise)


@functools.cache
def _make_sc_kernels():
    info = plsc.get_sparse_core_info()
    NC, NS = info.num_cores, info.num_subcores
    NW = NC * NS
    CH = N // NW
    mesh = plsc.VectorSubcoreMesh(core_axis_name="c", subcore_axis_name="s")

    @functools.partial(
        pl.kernel,
        out_type=(jax.ShapeDtypeStruct((P, D), jnp.float32),
                  jax.ShapeDtypeStruct((P, 128), jnp.float32)),
        mesh=mesh,
        scratch_types=[
            pltpu.VMEM((CH,), jnp.int32),
            pltpu.VMEM((CH,), jnp.int32),
            pltpu.VMEM((CH, D), jnp.float32),
            pltpu.VMEM((CH, 128), jnp.float32),
            pltpu.VMEM((CH, 128), jnp.float32),
            pltpu.SemaphoreType.DMA,
            pltpu.SemaphoreType.DMA,
        ],
    )
    def dispatch(x_hbm, dst0_hbm, dst1_hbm, w0_hbm, w1_hbm,
                 xs_hbm, ws_hbm,
                 idx0_v, idx1_v, rows_v, w0_v, w1_v, sem1, sem2):
        wid = lax.axis_index("s") * NC + lax.axis_index("c")
        base = wid * CH
        pltpu.sync_copy(dst0_hbm.at[pl.ds(base, CH)], idx0_v)
        pltpu.sync_copy(dst1_hbm.at[pl.ds(base, CH)], idx1_v)
        pltpu.sync_copy(x_hbm.at[pl.ds(base, CH)], rows_v)
        pltpu.sync_copy(w0_hbm.at[pl.ds(base, CH)], w0_v)
        pltpu.sync_copy(w1_hbm.at[pl.ds(base, CH)], w1_v)
        c1 = pltpu.async_copy(rows_v, xs_hbm.at[idx0_v], sem1)
        c2 = pltpu.async_copy(rows_v, xs_hbm.at[idx1_v], sem2)
        c1.wait()
        c2.wait()
        c3 = pltpu.async_copy(w0_v, ws_hbm.at[idx0_v], sem1)
        c4 = pltpu.async_copy(w1_v, ws_hbm.at[idx1_v], sem2)
        c3.wait()
        c4.wait()

    CH2 = CH // 2

    @functools.partial(
        pl.kernel,
        out_type=jax.ShapeDtypeStruct((N, D), jnp.float32),
        mesh=mesh,
        scratch_types=[
            pltpu.VMEM((CH2,), jnp.int32),
            pltpu.VMEM((CH2,), jnp.int32),
            pltpu.VMEM((CH2, D), jnp.float32),
            pltpu.VMEM((CH2, D), jnp.float32),
            pltpu.SemaphoreType.DMA,
            pltpu.SemaphoreType.DMA,
        ],
    )
    def combine(ys_hbm, dst0_hbm, dst1_hbm,
                out_hbm,
                idx0_v, idx1_v, r0_v, r1_v, sem1, sem2):
        wid = lax.axis_index("s") * NC + lax.axis_index("c")
        for h in range(2):
            base = wid * CH + h * CH2
            pltpu.sync_copy(dst0_hbm.at[pl.ds(base, CH2)], idx0_v)
            pltpu.sync_copy(dst1_hbm.at[pl.ds(base, CH2)], idx1_v)
            c1 = pltpu.async_copy(ys_hbm.at[idx0_v], r0_v, sem1)
            c2 = pltpu.async_copy(ys_hbm.at[idx1_v], r1_v, sem2)
            c1.wait()
            c2.wait()

            def row_add(i, carry):
                for j in range(D // 16):
                    s = pl.ds(j * 16, 16)
                    r0_v[i, s] = r0_v[i, s] + r1_v[i, s]
                return carry

            lax.fori_loop(0, CH2, row_add, 0)
            pltpu.sync_copy(r0_v, out_hbm.at[pl.ds(base, CH2)])

    return dispatch, combine


def _matmul_body(eot_sm, tval_sm, xs_ref, ws_ref, we_ref, be_ref, ys_ref):
    t = pl.program_id(0)

    @pl.when(tval_sm[t] != 0)
    def _():
        y = jnp.dot(xs_ref[...], we_ref[0], preferred_element_type=jnp.float32)
        ys_ref[...] = (y + be_ref[0]) * ws_ref[:, :1]


def _grouped_matmul(eot, tval, xs, ws, We, be):
    grid_spec = pltpu.PrefetchScalarGridSpec(
        num_scalar_prefetch=2,
        grid=(NT,),
        in_specs=[
            pl.BlockSpec((BM, D), lambda t, eot_sm, tval_sm: (t, 0)),
            pl.BlockSpec((BM, 128), lambda t, eot_sm, tval_sm: (t, 0)),
            pl.BlockSpec((1, D, D), lambda t, eot_sm, tval_sm: (eot_sm[t], 0, 0)),
            pl.BlockSpec((1, 1, D), lambda t, eot_sm, tval_sm: (eot_sm[t], 0, 0)),
        ],
        out_specs=pl.BlockSpec((BM, D), lambda t, eot_sm, tval_sm: (t, 0)),
    )
    return pl.pallas_call(
        _matmul_body,
        grid_spec=grid_spec,
        out_shape=jax.ShapeDtypeStruct((P, D), jnp.float32),
        compiler_params=pltpu.CompilerParams(
            dimension_semantics=("arbitrary",)),
    )(eot, tval, xs, ws, We, be.reshape(E, 1, D))


def _add_body(a_ref, b_ref, o_ref):
    o_ref[...] = a_ref[...] + b_ref[...]


def _add(a, b):
    nb = 8
    rb = N // nb
    return pl.pallas_call(
        _add_body,
        grid=(nb,),
        in_specs=[pl.BlockSpec((rb, D), lambda i: (i, 0)),
                  pl.BlockSpec((rb, D), lambda i: (i, 0))],
        out_specs=pl.BlockSpec((rb, D), lambda i: (i, 0)),
        out_shape=jax.ShapeDtypeStruct((N, D), jnp.float32),
    )(a, b)


@functools.cache
def _noise_const():
    return jax.random.normal(jax.random.key(42), (N, E), dtype=jnp.float32)


@functools.cache
def _wcat_cache():
    return None


def kernel(x, Wg, bg, Wn, bn, We, be):
    noise = _noise_const()
    wcat = jnp.concatenate([Wg, Wn], axis=1)
    bcat = jnp.concatenate([bg, bn]).reshape(1, 2 * E)

    gating, dst0c, dst1c, w0r, w1r, eot2, tval2 = _routing(x, wcat, bcat, noise)
    dst0 = dst0c.reshape(N)
    dst1 = dst1c.reshape(N)
    eot = eot2.reshape(NT)
    tval = tval2.reshape(NT)

    dispatch, combine = _make_sc_kernels()
    xs, ws = dispatch(x, dst0, dst1, w0r, w1r)
    ys = _grouped_matmul(eot, tval, xs, ws, We, be)
    updates = combine(ys, dst0, dst1)
    return updates, gating


# speedup vs baseline: 1.0790x; 1.0790x over previous
import functools

import jax
import jax.numpy as jnp
from jax import lax
from jax.experimental import pallas as pl
from jax.experimental.pallas import tpu as pltpu
from jax.experimental.pallas import tpu_sc as plsc

N = 2048
D = 1024
E = 8
K = 2
BM = 256
NT = (N * K) // BM + E
P = NT * BM
NEG = -1e30


def _routing_body(x_ref, wcat_ref, bcat_ref, noise_ref,
                  gating_ref, dst0_ref, dst1_ref, w0_ref, w1_ref,
                  eot_ref, tval_ref, tmap_ref):
    x = x_ref[...]
    z = jnp.dot(x, wcat_ref[...], preferred_element_type=jnp.float32)
    b = bcat_ref[...]
    logits = z[:, :E] + b[:, :E]
    nlog = z[:, E:] + b[:, E:]
    sp = jnp.maximum(nlog, 0.0) + jnp.log1p(jnp.exp(-jnp.abs(nlog)))
    noisy = logits + noise_ref[...] * sp

    iota8 = lax.broadcasted_iota(jnp.int32, (N, E), 1)
    v0 = jnp.max(noisy, axis=1, keepdims=True)
    c0 = jnp.where(noisy >= v0, iota8, E)
    i0 = jnp.min(c0, axis=1, keepdims=True)
    oh0 = iota8 == i0
    masked = jnp.where(oh0, NEG, noisy)
    v1 = jnp.max(masked, axis=1, keepdims=True)
    c1 = jnp.where(masked >= v1, iota8, E)
    i1 = jnp.min(c1, axis=1, keepdims=True)
    oh1 = iota8 == i1

    e1 = jnp.exp(v1 - v0)
    denom = 1.0 + e1
    g0 = 1.0 / denom
    g1 = e1 / denom
    gating_ref[...] = jnp.where(oh0, g0, 0.0) + jnp.where(oh1, g1, 0.0)
    w0_ref[...] = jnp.broadcast_to(g0, (N, 16))
    w1_ref[...] = jnp.broadcast_to(g1, (N, 16))

    maskf = (oh0 | oh1).astype(jnp.float32)

    RB = 256
    tri = (lax.broadcasted_iota(jnp.int32, (RB, RB), 1)
           < lax.broadcasted_iota(jnp.int32, (RB, RB), 0)).astype(jnp.float32)
    running = jnp.zeros((1, E), jnp.float32)
    blocks = []
    for bi in range(N // RB):
        mb = maskf[bi * RB:(bi + 1) * RB, :]
        blocks.append(jnp.dot(tri, mb, preferred_element_type=jnp.float32)
                      + running)
        running = running + jnp.sum(mb, axis=0, keepdims=True)
    rank = jnp.concatenate(blocks, axis=0)

    counts = running.astype(jnp.int32)
    tile_cnt = (counts + (BM - 1)) // BM
    lane = lax.broadcasted_iota(jnp.int32, (1, E), 1)
    tile_off = jnp.zeros((1, E), jnp.int32)
    for e in range(1, E):
        prev = jnp.sum(jnp.where(lane == e - 1, tile_cnt, 0),
                       axis=1, keepdims=True)
        tile_off = tile_off + jnp.where(lane >= e, prev, 0)
    row_off = (tile_off * BM).astype(jnp.float32)

    dstpos = jnp.broadcast_to(row_off, (N, E)) + rank
    dst0_ref[...] = jnp.sum(jnp.where(oh0, dstpos, 0.0), axis=1,
                            keepdims=True).astype(jnp.int32)
    dst1_ref[...] = jnp.sum(jnp.where(oh1, dstpos, 0.0), axis=1,
                            keepdims=True).astype(jnp.int32)

    ends = tile_off + tile_cnt
    nt_act = jnp.sum(tile_cnt, axis=1, keepdims=True)
    tt = lax.broadcasted_iota(jnp.int32, (NT, E), 0)
    eotv = jnp.sum((tt >= jnp.broadcast_to(ends, (NT, E))).astype(jnp.int32),
                   axis=1, keepdims=True)
    eot_ref[...] = jnp.minimum(eotv, E - 1)
    tval_ref[...] = (eotv < E).astype(jnp.int32)
    tcol = lax.broadcasted_iota(jnp.int32, (NT, 1), 0)
    tmap_ref[...] = jnp.minimum(tcol, jnp.broadcast_to(nt_act - 1, (NT, 1)))


def _routing(x, wcat, bcat, noise):
    return pl.pallas_call(
        _routing_body,
        out_shape=(
            jax.ShapeDtypeStruct((N, E), jnp.float32),
            jax.ShapeDtypeStruct((N, 1), jnp.int32),
            jax.ShapeDtypeStruct((N, 1), jnp.int32),
            jax.ShapeDtypeStruct((N, 16), jnp.float32),
            jax.ShapeDtypeStruct((N, 16), jnp.float32),
            jax.ShapeDtypeStruct((NT, 1), jnp.int32),
            jax.ShapeDtypeStruct((NT, 1), jnp.int32),
            jax.ShapeDtypeStruct((NT, 1), jnp.int32),
        ),
    )(x, wcat, bcat, noise)


@functools.cache
def _make_sc_kernels():
    info = plsc.get_sparse_core_info()
    NC, NS = info.num_cores, info.num_subcores
    NW = NC * NS
    CH = N // NW
    mesh = plsc.VectorSubcoreMesh(core_axis_name="c", subcore_axis_name="s")

    @functools.partial(
        pl.kernel,
        out_type=jax.ShapeDtypeStruct((P, D), jnp.float32),
        mesh=mesh,
        scratch_types=[
            pltpu.VMEM((CH,), jnp.int32),
            pltpu.VMEM((CH,), jnp.int32),
            pltpu.VMEM((CH, D), jnp.float32),
            pltpu.SemaphoreType.DMA,
            pltpu.SemaphoreType.DMA,
        ],
    )
    def dispatch(x_hbm, dst0_hbm, dst1_hbm,
                 xs_hbm,
                 idx0_v, idx1_v, rows_v, sem1, sem2):
        wid = lax.axis_index("s") * NC + lax.axis_index("c")
        base = wid * CH
        pltpu.sync_copy(dst0_hbm.at[pl.ds(base, CH)], idx0_v)
        pltpu.sync_copy(dst1_hbm.at[pl.ds(base, CH)], idx1_v)
        pltpu.sync_copy(x_hbm.at[pl.ds(base, CH)], rows_v)
        c1 = pltpu.async_copy(rows_v, xs_hbm.at[idx0_v], sem1)
        c2 = pltpu.async_copy(rows_v, xs_hbm.at[idx1_v], sem2)
        c1.wait()
        c2.wait()

    CH2 = CH // 2

    @functools.partial(
        pl.kernel,
        out_type=jax.ShapeDtypeStruct((N, D), jnp.float32),
        mesh=mesh,
        scratch_types=[
            pltpu.VMEM((CH2,), jnp.int32),
            pltpu.VMEM((CH2,), jnp.int32),
            pltpu.VMEM((CH2, 16), jnp.float32),
            pltpu.VMEM((CH2, 16), jnp.float32),
            pltpu.VMEM((CH2, D), jnp.float32),
            pltpu.VMEM((CH2, D), jnp.float32),
            pltpu.SemaphoreType.DMA,
            pltpu.SemaphoreType.DMA,
        ],
    )
    def combine(ys_hbm, dst0_hbm, dst1_hbm, w0_hbm, w1_hbm,
                out_hbm,
                idx0_v, idx1_v, w0_v, w1_v, r0_v, r1_v, sem1, sem2):
        wid = lax.axis_index("s") * NC + lax.axis_index("c")
        for h in range(2):
            base = wid * CH + h * CH2
            pltpu.sync_copy(dst0_hbm.at[pl.ds(base, CH2)], idx0_v)
            pltpu.sync_copy(dst1_hbm.at[pl.ds(base, CH2)], idx1_v)
            pltpu.sync_copy(w0_hbm.at[pl.ds(base, CH2)], w0_v)
            pltpu.sync_copy(w1_hbm.at[pl.ds(base, CH2)], w1_v)
            c1 = pltpu.async_copy(ys_hbm.at[idx0_v], r0_v, sem1)
            c2 = pltpu.async_copy(ys_hbm.at[idx1_v], r1_v, sem2)
            c1.wait()
            c2.wait()

            def row_add(i, carry):
                ws0 = w0_v[i, :]
                ws1 = w1_v[i, :]
                for j in range(D // 16):
                    s = pl.ds(j * 16, 16)
                    r0_v[i, s] = ws0 * r0_v[i, s] + ws1 * r1_v[i, s]
                return carry

            lax.fori_loop(0, CH2, row_add, 0)
            pltpu.sync_copy(r0_v, out_hbm.at[pl.ds(base, CH2)])

    return dispatch, combine


def _matmul_body(eot_sm, tval_sm, tmap_sm, xs_ref, we_ref, be_ref, ys_ref):
    t = pl.program_id(0)

    @pl.when(tval_sm[t] != 0)
    def _():
        y = jnp.dot(xs_ref[...], we_ref[0], preferred_element_type=jnp.float32)
        ys_ref[...] = y + be_ref[0]


def _grouped_matmul(eot, tval, tmap, xs, We, be):
    grid_spec = pltpu.PrefetchScalarGridSpec(
        num_scalar_prefetch=3,
        grid=(NT,),
        in_specs=[
            pl.BlockSpec((BM, D), lambda t, eot_sm, tval_sm, tmap_sm:
                         (tmap_sm[t], 0)),
            pl.BlockSpec((1, D, D), lambda t, eot_sm, tval_sm, tmap_sm:
                         (eot_sm[t], 0, 0)),
            pl.BlockSpec((1, 1, D), lambda t, eot_sm, tval_sm, tmap_sm:
                         (eot_sm[t], 0, 0)),
        ],
        out_specs=pl.BlockSpec((BM, D), lambda t, eot_sm, tval_sm, tmap_sm:
                               (tmap_sm[t], 0)),
    )
    return pl.pallas_call(
        _matmul_body,
        grid_spec=grid_spec,
        out_shape=jax.ShapeDtypeStruct((P, D), jnp.float32),
        compiler_params=pltpu.CompilerParams(
            dimension_semantics=("arbitrary",)),
    )(eot, tval, tmap, xs, We, be.reshape(E, 1, D))


@functools.cache
def _noise_const():
    return jax.random.normal(jax.random.key(42), (N, E), dtype=jnp.float32)


def kernel(x, Wg, bg, Wn, bn, We, be):
    noise = _noise_const()
    wcat = jnp.concatenate([Wg, Wn], axis=1)
    bcat = jnp.concatenate([bg, bn]).reshape(1, 2 * E)

    gating, dst0c, dst1c, w0, w1, eot2, tval2, tmap2 = _routing(
        x, wcat, bcat, noise)
    dst0 = dst0c.reshape(N)
    dst1 = dst1c.reshape(N)
    eot = eot2.reshape(NT)
    tval = tval2.reshape(NT)
    tmap = tmap2.reshape(NT)

    dispatch, combine = _make_sc_kernels()
    xs = dispatch(x, dst0, dst1)
    ys = _grouped_matmul(eot, tval, tmap, xs, We, be)
    updates = combine(ys, dst0, dst1, w0, w1)
    return updates, gating
